# baseline (speedup 1.0000x reference)
import jax
import jax.numpy as jnp
from jax.experimental import pallas as pl
from jax.experimental.pallas import tpu as pltpu

_M_BLK = 400


def _gcn_body(adj_ref, x_ref, w1_ref, b1_ref, w2_ref, b2_ref, out_ref,
              s1_ref, s2_ref):
    p = pl.program_id(0)
    i = pl.program_id(1)

    @pl.when((p == 0) & (i == 0))
    def _():
        s1_ref[...] = jnp.dot(
            x_ref[...], w1_ref[...], preferred_element_type=jnp.float32
        ).astype(jnp.bfloat16)

    a16 = adj_ref[...].astype(jnp.bfloat16)

    @pl.when(p == 0)
    def _():
        h = jnp.dot(a16, s1_ref[...], preferred_element_type=jnp.float32)
        h = jnp.maximum(h + b1_ref[...], 0.0)
        s2_ref[pl.ds(i * _M_BLK, _M_BLK), :] = jnp.dot(
            h, w2_ref[...], preferred_element_type=jnp.float32
        ).astype(jnp.bfloat16)

    @pl.when(p == 1)
    def _():
        h = jnp.dot(a16, s2_ref[...], preferred_element_type=jnp.float32)
        out_ref[...] = jnp.maximum(h + b2_ref[...], 0.0)


def kernel(x, adj, W1, b1, W2, b2):
    n, f = adj.shape[0], x.shape[1]
    b1r = b1.reshape(1, -1)
    b2r = b2.reshape(1, -1)

    return pl.pallas_call(
        _gcn_body,
        grid=(2, n // _M_BLK),
        in_specs=[
            pl.BlockSpec((_M_BLK, n), lambda p, i: (i, 0)),
            pl.BlockSpec((n, f), lambda p, i: (0, 0)),
            pl.BlockSpec((f, f), lambda p, i: (0, 0)),
            pl.BlockSpec((1, f), lambda p, i: (0, 0)),
            pl.BlockSpec((f, f), lambda p, i: (0, 0)),
            pl.BlockSpec((1, f), lambda p, i: (0, 0)),
        ],
        out_specs=pl.BlockSpec((_M_BLK, f), lambda p, i: (p * i, 0)),
        out_shape=jax.ShapeDtypeStruct((n, f), jnp.float32),
        scratch_shapes=[
            pltpu.VMEM((n, f), jnp.bfloat16),
            pltpu.VMEM((n, f), jnp.bfloat16),
        ],
        compiler_params=pltpu.CompilerParams(
            dimension_semantics=("arbitrary", "arbitrary"),
        ),
    )(adj, x, W1, b1r, W2, b2r)


# speedup vs baseline: 1.0201x; 1.0201x over previous
import jax
import jax.numpy as jnp
from jax.experimental import pallas as pl
from jax.experimental.pallas import tpu as pltpu

_M_BLK = 400


def _gcn_body(adj_ref, x_ref, w1_ref, b1_ref, w2_ref, b2_ref, out_ref,
              s1_ref, s2_ref):
    p = pl.program_id(0)
    i = pl.program_id(1)

    @pl.when((p == 0) & (i == 0))
    def _():
        s1_ref[...] = jnp.dot(
            x_ref[...], w1_ref[...], preferred_element_type=jnp.float32
        ).astype(jnp.bfloat16)

    @pl.when(p == 0)
    def _():
        h = jax.lax.dot_general(
            adj_ref[...], s1_ref[...], (((1,), (0,)), ((), ())),
            preferred_element_type=jnp.float32)
        h = jnp.maximum(h + b1_ref[...], 0.0)
        s2_ref[pl.ds(i * _M_BLK, _M_BLK), :] = jnp.dot(
            h, w2_ref[...], preferred_element_type=jnp.float32
        ).astype(jnp.bfloat16)

    @pl.when(p == 1)
    def _():
        h = jax.lax.dot_general(
            adj_ref[...], s2_ref[...], (((1,), (0,)), ((), ())),
            preferred_element_type=jnp.float32)
        out_ref[...] = jnp.maximum(h + b2_ref[...], 0.0)


def kernel(x, adj, W1, b1, W2, b2):
    n, f = adj.shape[0], x.shape[1]
    b1r = b1.reshape(1, -1)
    b2r = b2.reshape(1, -1)

    return pl.pallas_call(
        _gcn_body,
        grid=(2, n // _M_BLK),
        in_specs=[
            pl.BlockSpec((_M_BLK, n), lambda p, i: (i, 0)),
            pl.BlockSpec((n, f), lambda p, i: (0, 0)),
            pl.BlockSpec((f, f), lambda p, i: (0, 0)),
            pl.BlockSpec((1, f), lambda p, i: (0, 0)),
            pl.BlockSpec((f, f), lambda p, i: (0, 0)),
            pl.BlockSpec((1, f), lambda p, i: (0, 0)),
        ],
        out_specs=pl.BlockSpec((_M_BLK, f), lambda p, i: (p * i, 0)),
        out_shape=jax.ShapeDtypeStruct((n, f), jnp.float32),
        scratch_shapes=[
            pltpu.VMEM((n, f), jnp.bfloat16),
            pltpu.VMEM((n, f), jnp.bfloat16),
        ],
        compiler_params=pltpu.CompilerParams(
            dimension_semantics=("arbitrary", "arbitrary"),
        ),
    )(adj, x, W1, b1r, W2, b2r)
